# baseline (speedup 1.0000x reference)
import functools

import jax
import jax.numpy as jnp
from jax import lax
from jax.experimental import pallas as pl
from jax.experimental.pallas import tpu as pltpu
from jax.experimental.pallas import tpu_sc as plsc

N = 10000
NPAD = 10240
D = 256
HW = D // 2
L = 5
OUT = 3
EPS = 1e-5
BR = 1024

NC = 2
NS = 16
CH = 128
E = 160000
EPT = 10240
NCHUNK = EPT // CH
EPAD = EPT * NS
ZR = 64

_mesh = plsc.VectorSubcoreMesh(core_axis_name="c", subcore_axis_name="s",
                               num_cores=NC, num_subcores=NS)


EPW = EPAD // (NC * NS)
NCHD = EPW // CH


@functools.partial(
    pl.kernel,
    out_type=jax.ShapeDtypeStruct((2 * NPAD, HW), jnp.float32),
    mesh=_mesh,
    scratch_types=[
        pltpu.VMEM((CH,), jnp.int32),
        pltpu.VMEM((CH,), jnp.int32),
        pltpu.VMEM((CH, HW), jnp.float32),
        pltpu.VMEM((ZR, HW), jnp.float32),
        pltpu.VMEM_SHARED((NPAD, HW), jnp.float32),
        pltpu.SemaphoreType.DMA,
        pltpu.SemaphoreType.DMA,
    ],
)
def _sc_degree(colp_hbm, ones_hbm, zeros_hbm, deg_hbm,
               iA, iB, ones, zbuf, spdeg, sA, sB):
    s = lax.axis_index("c")
    t = lax.axis_index("s")
    pltpu.sync_copy(ones_hbm, ones)
    pltpu.sync_copy(zeros_hbm, zbuf)
    rpt = NPAD // NS
    for j in range(rpt // ZR):
        pltpu.sync_copy(zbuf, spdeg.at[pl.ds(t * rpt + j * ZR, ZR)])
    plsc.subcore_barrier()
    base = (s * NS + t) * EPW

    def wait_scatter(sem):
        pltpu.make_async_copy(ones_hbm, ones, sem).wait()

    pltpu.sync_copy(colp_hbm.at[pl.ds(base, CH)], iA)
    pltpu.async_copy(ones, spdeg.at[iA], sA, add=True)
    pltpu.sync_copy(colp_hbm.at[pl.ds(base + CH, CH)], iB)
    pltpu.async_copy(ones, spdeg.at[iB], sB, add=True)

    def pair(k2, _):
        a = 2 * k2
        wait_scatter(sA)
        pltpu.sync_copy(colp_hbm.at[pl.ds(base + (a + 2) * CH, CH)], iA)
        pltpu.async_copy(ones, spdeg.at[iA], sA, add=True)
        wait_scatter(sB)
        pltpu.sync_copy(colp_hbm.at[pl.ds(base + (a + 3) * CH, CH)], iB)
        pltpu.async_copy(ones, spdeg.at[iB], sB, add=True)
        return 0

    lax.fori_loop(0, NCHD // 2 - 1, pair, 0)
    wait_scatter(sA)
    wait_scatter(sB)
    plsc.subcore_barrier()
    pltpu.sync_copy(spdeg.at[pl.ds(t * rpt, rpt)],
                    deg_hbm.at[pl.ds(s * NPAD + t * rpt, rpt)])


NQ = NCHUNK // 4


@functools.partial(
    pl.kernel,
    out_type=jax.ShapeDtypeStruct((2 * NPAD, HW), jnp.float32),
    mesh=_mesh,
    scratch_types=[
        pltpu.VMEM((CH,), jnp.int32),
        pltpu.VMEM((CH,), jnp.int32),
        pltpu.VMEM((CH,), jnp.int32),
        pltpu.VMEM((CH,), jnp.int32),
        pltpu.VMEM((CH,), jnp.int32),
        pltpu.VMEM((CH,), jnp.int32),
        pltpu.VMEM((CH,), jnp.int32),
        pltpu.VMEM((CH,), jnp.int32),
        pltpu.VMEM((CH, HW), jnp.float32),
        pltpu.VMEM((CH, HW), jnp.float32),
        pltpu.VMEM((ZR, HW), jnp.float32),
        pltpu.VMEM_SHARED((NPAD, HW), jnp.float32),
        pltpu.SemaphoreType.DMA,
        pltpu.SemaphoreType.DMA,
        pltpu.SemaphoreType.DMA,
        pltpu.SemaphoreType.DMA,
        pltpu.SemaphoreType.DMA,
        pltpu.SemaphoreType.DMA,
    ],
)
def _sc_segsum(zs_hbm, rows2_hbm, cols2_hbm, zeros_hbm, accs_hbm,
               Ar0, Ac0, Ar1, Ac1, Br0, Bc0, Br1, Bc1, gA, gB, zbuf, spacc,
               sGA, sGB, sSA, sSB, sIA, sIB):
    s = lax.axis_index("c")
    t = lax.axis_index("s")
    pltpu.sync_copy(zeros_hbm, zbuf)
    rpt = NPAD // NS
    for j in range(rpt // ZR):
        pltpu.sync_copy(zbuf, spacc.at[pl.ds(t * rpt + j * ZR, ZR)])
    plsc.subcore_barrier()
    rbase = (s * NS + t) * EPT
    cbase = t * EPT

    def idx_sync(k, ir, ic):
        pltpu.sync_copy(rows2_hbm.at[pl.ds(rbase + k * CH, CH)], ir)
        pltpu.sync_copy(cols2_hbm.at[pl.ds(cbase + k * CH, CH)], ic)

    def idx_async(k, ir, ic, sem):
        pltpu.async_copy(rows2_hbm.at[pl.ds(rbase + k * CH, CH)], ir, sem)
        pltpu.async_copy(cols2_hbm.at[pl.ds(cbase + k * CH, CH)], ic, sem)

    def wait_g(g, sem):
        pltpu.make_async_copy(zs_hbm.at[pl.ds(0, CH)], g, sem).wait()

    def wait_idx(ir, ic, sem):
        pltpu.make_async_copy(rows2_hbm.at[pl.ds(0, CH)], ir, sem).wait()
        pltpu.make_async_copy(rows2_hbm.at[pl.ds(0, CH)], ic, sem).wait()

    idx_sync(0, Ar0, Ac0)
    pltpu.async_copy(zs_hbm.at[Ar0], gA, sGA)
    idx_sync(1, Br0, Bc0)
    pltpu.async_copy(zs_hbm.at[Br0], gB, sGB)
    idx_async(2, Ar1, Ac1, sIA)
    idx_async(3, Br1, Bc1, sIB)

    def quad(k, _):
        q = 4 * k
        wait_g(gA, sGA)
        pltpu.async_copy(gA, spacc.at[Ac0], sSA, add=True)
        wait_g(gB, sGB)
        pltpu.async_copy(gB, spacc.at[Bc0], sSB, add=True)
        wait_idx(Ar1, Ac1, sIA)
        wait_g(gA, sSA)
        pltpu.async_copy(zs_hbm.at[Ar1], gA, sGA)
        idx_async(q + 4, Ar0, Ac0, sIA)
        wait_idx(Br1, Bc1, sIB)
        wait_g(gB, sSB)
        pltpu.async_copy(zs_hbm.at[Br1], gB, sGB)
        idx_async(q + 5, Br0, Bc0, sIB)
        wait_g(gA, sGA)
        pltpu.async_copy(gA, spacc.at[Ac1], sSA, add=True)
        wait_g(gB, sGB)
        pltpu.async_copy(gB, spacc.at[Bc1], sSB, add=True)
        wait_idx(Ar0, Ac0, sIA)
        wait_g(gA, sSA)
        pltpu.async_copy(zs_hbm.at[Ar0], gA, sGA)
        idx_async(q + 6, Ar1, Ac1, sIA)
        wait_idx(Br0, Bc0, sIB)
        wait_g(gB, sSB)
        pltpu.async_copy(zs_hbm.at[Br0], gB, sGB)
        idx_async(q + 7, Br1, Bc1, sIB)
        return 0

    lax.fori_loop(0, NQ - 1, quad, 0)
    wait_g(gA, sGA)
    pltpu.sync_copy(gA, spacc.at[Ac0], add=True)
    wait_idx(Ar1, Ac1, sIA)
    pltpu.async_copy(zs_hbm.at[Ar1], gA, sGA)
    wait_g(gB, sGB)
    pltpu.sync_copy(gB, spacc.at[Bc0], add=True)
    wait_idx(Br1, Bc1, sIB)
    pltpu.async_copy(zs_hbm.at[Br1], gB, sGB)
    wait_g(gA, sGA)
    pltpu.sync_copy(gA, spacc.at[Ac1], add=True)
    wait_g(gB, sGB)
    pltpu.sync_copy(gB, spacc.at[Bc1], add=True)
    plsc.subcore_barrier()
    pltpu.sync_copy(spacc.at[pl.ds(t * rpt, rpt)],
                    accs_hbm.at[pl.ds(s * NPAD + t * rpt, rpt)])


def _dis(dga_ref, dgb_ref):
    return lax.rsqrt(dga_ref[:, 0:1] + dgb_ref[:, 0:1] + 1.0)


def _tc_first_body(h_ref, w_ref, dga_ref, dgb_ref, z_ref):
    z_ref[...] = _dis(dga_ref, dgb_ref) * jnp.dot(
        h_ref[...], w_ref[...], preferred_element_type=jnp.float32)


def _epilogue(accl_ref, accr_ref, zp_ref, dga_ref, dgb_ref,
              b_ref, gs_ref, beta_ref):
    dis = _dis(dga_ref, dgb_ref)
    zp = zp_ref[...]
    conv = dis * jnp.concatenate(
        [accl_ref[...] + zp[:, 0:HW], accr_ref[...] + zp[:, HW:D]], axis=1)
    conv = conv + b_ref[0:1, :]
    a = jnp.where(conv > 0, conv, 0.2 * conv)
    return a * gs_ref[0:1, :] + beta_ref[0:1, :], dis


def _tc_mid_body(accl_ref, accr_ref, zp_ref, hp_ref, w_ref, dga_ref,
                 dgb_ref, b_ref, gs_ref, beta_ref, h_ref, z_ref):
    a, dis = _epilogue(accl_ref, accr_ref, zp_ref, dga_ref, dgb_ref,
                       b_ref, gs_ref, beta_ref)
    h = a + hp_ref[...]
    h_ref[...] = h
    z_ref[...] = dis * jnp.dot(h, w_ref[...],
                               preferred_element_type=jnp.float32)


def _tc_last_body(accl_ref, accr_ref, zp_ref, hp_ref, dga_ref, dgb_ref,
                  b_ref, gs_ref, beta_ref, z_ref):
    a, dis = _epilogue(accl_ref, accr_ref, zp_ref, dga_ref, dgb_ref,
                       b_ref, gs_ref, beta_ref)
    z_ref[...] = dis * (a + hp_ref[...])


def _tc_out_body(accl_ref, accr_ref, zp_ref, dga_ref, dgb_ref,
                 w_ref, b_ref, dx_ref):
    dis = _dis(dga_ref, dgb_ref)
    zp = zp_ref[...]
    agg = dis * jnp.concatenate(
        [accl_ref[...] + zp[:, 0:HW], accr_ref[...] + zp[:, HW:D]], axis=1)
    dx_ref[...] = jnp.dot(agg, w_ref[...],
                          preferred_element_type=jnp.float32) + b_ref[0:1, :]


def _row_spec(width=D):
    return pl.BlockSpec((BR, width), lambda i: (i, 0))


_accl_spec = pl.BlockSpec((BR, HW), lambda i: (i, 0))
_accr_spec = pl.BlockSpec((BR, HW), lambda i: (i + NPAD // BR, 0))


def _full_spec(shape):
    return pl.BlockSpec(shape, lambda i: (0, 0))


def _pcall(body, in_specs, out_widths):
    outs = tuple(jax.ShapeDtypeStruct((NPAD, w), jnp.float32)
                 for w in out_widths)
    out_specs = tuple(_row_spec(w) for w in out_widths)
    if len(out_widths) == 1:
        outs, out_specs = outs[0], out_specs[0]
    return pl.pallas_call(body, grid=(NPAD // BR,), in_specs=in_specs,
                          out_specs=out_specs, out_shape=outs)


def kernel(x, edge_index, W, b, gamma, beta, W_out, b_out):
    row = edge_index[0]
    col = edge_index[1]
    pr = (jnp.arange(EPAD - E, dtype=jnp.int32) * 37) % N
    pc = N + (jnp.arange(EPAD - E, dtype=jnp.int32) % (NPAD - N))
    rowp = jnp.concatenate([row, pr])
    colp = jnp.concatenate([col, pc])
    rp2 = rowp * 2
    rows2 = jnp.concatenate([rp2, rp2 + 1])
    cols2 = colp
    xp = jnp.pad(x, ((0, NPAD - N), (0, 0)))
    Wt = jnp.transpose(W, (0, 2, 1))
    scale = 1.0 / jnp.sqrt(1.0 + EPS)
    gs = gamma * scale
    b8 = jnp.broadcast_to(b[:, None, :], (L, 8, D))
    gs8 = jnp.broadcast_to(gs[:, None, :], (L, 8, D))
    beta8 = jnp.broadcast_to(beta[:, None, :], (L, 8, D))
    wof = jnp.zeros((D, 128), jnp.float32).at[:, :OUT].set(W_out.T)
    bof = jnp.broadcast_to(jnp.pad(b_out, (0, 128 - OUT))[None, :], (8, 128))

    ones128 = jnp.ones((CH, HW), jnp.float32)
    zeroshw = jnp.zeros((ZR, HW), jnp.float32)

    deg2 = _sc_degree(colp, ones128, zeroshw)

    par_spec = _full_spec((8, D))

    def seg(z):
        return _sc_segsum(z.reshape(2 * NPAD, HW), rows2, cols2, zeroshw)

    z = _pcall(_tc_first_body,
               [_row_spec(), _full_spec((D, D)), _accl_spec, _accr_spec],
               (D,))(xp, Wt[0], deg2, deg2)
    h = xp
    for i in range(1, L):
        accs = seg(z)
        h, z = _pcall(_tc_mid_body,
                      [_accl_spec, _accr_spec, _row_spec(), _row_spec(),
                       _full_spec((D, D)), _accl_spec, _accr_spec,
                       par_spec, par_spec, par_spec], (D, D))(
            accs, accs, z, h, Wt[i], deg2, deg2,
            b8[i - 1], gs8[i - 1], beta8[i - 1])
    accs = seg(z)
    z = _pcall(_tc_last_body,
               [_accl_spec, _accr_spec, _row_spec(), _row_spec(),
                _accl_spec, _accr_spec,
                par_spec, par_spec, par_spec], (D,))(
        accs, accs, z, h, deg2, deg2, b8[L - 1], gs8[L - 1], beta8[L - 1])
    accs = seg(z)
    dxp = _pcall(_tc_out_body,
                 [_accl_spec, _accr_spec, _row_spec(), _accl_spec, _accr_spec,
                  _full_spec((D, 128)), _full_spec((8, 128))], (128,))(
        accs, accs, z, deg2, deg2, wof, bof)
    return dxp[:N, :OUT]


# speedup vs baseline: 1.1224x; 1.1224x over previous
import functools

import jax
import jax.numpy as jnp
from jax import lax
from jax.experimental import pallas as pl
from jax.experimental.pallas import tpu as pltpu
from jax.experimental.pallas import tpu_sc as plsc

N = 10000
NPAD = 10240
D = 256
HW = D // 2
L = 5
OUT = 3
EPS = 1e-5
BR = 1024

NC = 2
NS = 16
CH = 128
E = 160000
EPT = 10240
NCHUNK = EPT // CH
EPAD = EPT * NS
ZR = 64

_mesh = plsc.VectorSubcoreMesh(core_axis_name="c", subcore_axis_name="s",
                               num_cores=NC, num_subcores=NS)


EPW = EPAD // (NC * NS)
NCHD = EPW // CH


@functools.partial(
    pl.kernel,
    out_type=jax.ShapeDtypeStruct((2 * NPAD, HW), jnp.float32),
    mesh=_mesh,
    scratch_types=[
        pltpu.VMEM((CH,), jnp.int32),
        pltpu.VMEM((CH,), jnp.int32),
        pltpu.VMEM((CH, HW), jnp.float32),
        pltpu.VMEM((ZR, HW), jnp.float32),
        pltpu.VMEM_SHARED((NPAD, HW), jnp.float32),
        pltpu.SemaphoreType.DMA,
        pltpu.SemaphoreType.DMA,
    ],
)
def _sc_degree(colp_hbm, ones_hbm, zeros_hbm, deg_hbm,
               iA, iB, ones, zbuf, spdeg, sA, sB):
    s = lax.axis_index("c")
    t = lax.axis_index("s")
    pltpu.sync_copy(ones_hbm, ones)
    pltpu.sync_copy(zeros_hbm, zbuf)
    rpt = NPAD // NS
    for j in range(rpt // ZR):
        pltpu.sync_copy(zbuf, spdeg.at[pl.ds(t * rpt + j * ZR, ZR)])
    plsc.subcore_barrier()
    base = (s * NS + t) * EPW

    def wait_scatter(sem):
        pltpu.make_async_copy(ones_hbm, ones, sem).wait()

    pltpu.sync_copy(colp_hbm.at[pl.ds(base, CH)], iA)
    pltpu.async_copy(ones, spdeg.at[iA], sA, add=True)
    pltpu.sync_copy(colp_hbm.at[pl.ds(base + CH, CH)], iB)
    pltpu.async_copy(ones, spdeg.at[iB], sB, add=True)

    def pair(k2, _):
        a = 2 * k2
        wait_scatter(sA)
        pltpu.sync_copy(colp_hbm.at[pl.ds(base + (a + 2) * CH, CH)], iA)
        pltpu.async_copy(ones, spdeg.at[iA], sA, add=True)
        wait_scatter(sB)
        pltpu.sync_copy(colp_hbm.at[pl.ds(base + (a + 3) * CH, CH)], iB)
        pltpu.async_copy(ones, spdeg.at[iB], sB, add=True)
        return 0

    lax.fori_loop(0, NCHD // 2 - 1, pair, 0)
    wait_scatter(sA)
    wait_scatter(sB)
    plsc.subcore_barrier()
    pltpu.sync_copy(spdeg.at[pl.ds(t * rpt, rpt)],
                    deg_hbm.at[pl.ds(s * NPAD + t * rpt, rpt)])


NQ = NCHUNK // 4


@functools.partial(
    pl.kernel,
    out_type=jax.ShapeDtypeStruct((2 * NPAD, HW), jnp.float32),
    mesh=_mesh,
    scratch_types=[
        pltpu.VMEM((CH,), jnp.int32),
        pltpu.VMEM((CH,), jnp.int32),
        pltpu.VMEM((CH,), jnp.int32),
        pltpu.VMEM((CH,), jnp.int32),
        pltpu.VMEM((CH,), jnp.int32),
        pltpu.VMEM((CH,), jnp.int32),
        pltpu.VMEM((CH,), jnp.int32),
        pltpu.VMEM((CH,), jnp.int32),
        pltpu.VMEM((CH, HW), jnp.float32),
        pltpu.VMEM((CH, HW), jnp.float32),
        pltpu.VMEM((ZR, HW), jnp.float32),
        pltpu.VMEM_SHARED((NPAD, HW), jnp.float32),
        pltpu.SemaphoreType.DMA,
        pltpu.SemaphoreType.DMA,
        pltpu.SemaphoreType.DMA,
        pltpu.SemaphoreType.DMA,
    ],
)
def _sc_segsum(zs_hbm, rows2_hbm, cols2_hbm, zeros_hbm, accs_hbm,
               Ar0, Ac0, Ar1, Ac1, Br0, Bc0, Br1, Bc1, gA, gB, zbuf, spacc,
               sGA, sGB, sSA, sSB):
    s = lax.axis_index("c")
    t = lax.axis_index("s")
    pltpu.sync_copy(zeros_hbm, zbuf)
    rpt = NPAD // NS
    for j in range(rpt // ZR):
        pltpu.sync_copy(zbuf, spacc.at[pl.ds(t * rpt + j * ZR, ZR)])
    plsc.subcore_barrier()
    rbase = (s * NS + t) * EPT
    cbase = t * EPT

    def idx_sync(k, ir, ic):
        pltpu.sync_copy(rows2_hbm.at[pl.ds(rbase + k * CH, CH)], ir)
        pltpu.sync_copy(cols2_hbm.at[pl.ds(cbase + k * CH, CH)], ic)

    def wait_g(g, sem):
        pltpu.make_async_copy(zs_hbm.at[pl.ds(0, CH)], g, sem).wait()

    idx_sync(0, Ar0, Ac0)
    pltpu.async_copy(zs_hbm.at[Ar0], gA, sGA)
    idx_sync(1, Br0, Bc0)
    pltpu.async_copy(zs_hbm.at[Br0], gB, sGB)

    def quad(k, _):
        q = 4 * k
        wait_g(gA, sGA)
        pltpu.async_copy(gA, spacc.at[Ac0], sSA, add=True)
        idx_sync(q + 2, Ar1, Ac1)
        wait_g(gA, sSA)
        pltpu.async_copy(zs_hbm.at[Ar1], gA, sGA)
        wait_g(gB, sGB)
        pltpu.async_copy(gB, spacc.at[Bc0], sSB, add=True)
        idx_sync(q + 3, Br1, Bc1)
        wait_g(gB, sSB)
        pltpu.async_copy(zs_hbm.at[Br1], gB, sGB)
        wait_g(gA, sGA)
        pltpu.async_copy(gA, spacc.at[Ac1], sSA, add=True)
        idx_sync(q + 4, Ar0, Ac0)
        wait_g(gA, sSA)
        pltpu.async_copy(zs_hbm.at[Ar0], gA, sGA)
        wait_g(gB, sGB)
        pltpu.async_copy(gB, spacc.at[Bc1], sSB, add=True)
        idx_sync(q + 5, Br0, Bc0)
        wait_g(gB, sSB)
        pltpu.async_copy(zs_hbm.at[Br0], gB, sGB)
        return 0

    lax.fori_loop(0, NQ - 1, quad, 0)
    wait_g(gA, sGA)
    pltpu.sync_copy(gA, spacc.at[Ac0], add=True)
    idx_sync(NCHUNK - 2, Ar1, Ac1)
    pltpu.async_copy(zs_hbm.at[Ar1], gA, sGA)
    wait_g(gB, sGB)
    pltpu.sync_copy(gB, spacc.at[Bc0], add=True)
    idx_sync(NCHUNK - 1, Br1, Bc1)
    pltpu.async_copy(zs_hbm.at[Br1], gB, sGB)
    wait_g(gA, sGA)
    pltpu.sync_copy(gA, spacc.at[Ac1], add=True)
    wait_g(gB, sGB)
    pltpu.sync_copy(gB, spacc.at[Bc1], add=True)
    plsc.subcore_barrier()
    pltpu.sync_copy(spacc.at[pl.ds(t * rpt, rpt)],
                    accs_hbm.at[pl.ds(s * NPAD + t * rpt, rpt)])


def _dis(dga_ref, dgb_ref):
    return lax.rsqrt(dga_ref[:, 0:1] + dgb_ref[:, 0:1] + 1.0)


def _tc_first_body(h_ref, w_ref, dga_ref, dgb_ref, z_ref):
    z_ref[...] = _dis(dga_ref, dgb_ref) * jnp.dot(
        h_ref[...], w_ref[...], preferred_element_type=jnp.float32)


def _epilogue(accl_ref, accr_ref, zp_ref, dga_ref, dgb_ref,
              b_ref, gs_ref, beta_ref):
    dis = _dis(dga_ref, dgb_ref)
    zp = zp_ref[...]
    conv = dis * jnp.concatenate(
        [accl_ref[...] + zp[:, 0:HW], accr_ref[...] + zp[:, HW:D]], axis=1)
    conv = conv + b_ref[0:1, :]
    a = jnp.where(conv > 0, conv, 0.2 * conv)
    return a * gs_ref[0:1, :] + beta_ref[0:1, :], dis


def _tc_mid_body(accl_ref, accr_ref, zp_ref, hp_ref, w_ref, dga_ref,
                 dgb_ref, b_ref, gs_ref, beta_ref, h_ref, z_ref):
    a, dis = _epilogue(accl_ref, accr_ref, zp_ref, dga_ref, dgb_ref,
                       b_ref, gs_ref, beta_ref)
    h = a + hp_ref[...]
    h_ref[...] = h
    z_ref[...] = dis * jnp.dot(h, w_ref[...],
                               preferred_element_type=jnp.float32)


def _tc_last_body(accl_ref, accr_ref, zp_ref, hp_ref, dga_ref, dgb_ref,
                  b_ref, gs_ref, beta_ref, z_ref):
    a, dis = _epilogue(accl_ref, accr_ref, zp_ref, dga_ref, dgb_ref,
                       b_ref, gs_ref, beta_ref)
    z_ref[...] = dis * (a + hp_ref[...])


def _tc_out_body(accl_ref, accr_ref, zp_ref, dga_ref, dgb_ref,
                 w_ref, b_ref, dx_ref):
    dis = _dis(dga_ref, dgb_ref)
    zp = zp_ref[...]
    agg = dis * jnp.concatenate(
        [accl_ref[...] + zp[:, 0:HW], accr_ref[...] + zp[:, HW:D]], axis=1)
    dx_ref[...] = jnp.dot(agg, w_ref[...],
                          preferred_element_type=jnp.float32) + b_ref[0:1, :]


def _row_spec(width=D):
    return pl.BlockSpec((BR, width), lambda i: (i, 0))


_accl_spec = pl.BlockSpec((BR, HW), lambda i: (i, 0))
_accr_spec = pl.BlockSpec((BR, HW), lambda i: (i + NPAD // BR, 0))


def _full_spec(shape):
    return pl.BlockSpec(shape, lambda i: (0, 0))


def _pcall(body, in_specs, out_widths):
    outs = tuple(jax.ShapeDtypeStruct((NPAD, w), jnp.float32)
                 for w in out_widths)
    out_specs = tuple(_row_spec(w) for w in out_widths)
    if len(out_widths) == 1:
        outs, out_specs = outs[0], out_specs[0]
    return pl.pallas_call(body, grid=(NPAD // BR,), in_specs=in_specs,
                          out_specs=out_specs, out_shape=outs)


def kernel(x, edge_index, W, b, gamma, beta, W_out, b_out):
    row = edge_index[0]
    col = edge_index[1]
    pr = (jnp.arange(EPAD - E, dtype=jnp.int32) * 37) % N
    pc = N + (jnp.arange(EPAD - E, dtype=jnp.int32) % (NPAD - N))
    rowp = jnp.concatenate([row, pr])
    colp = jnp.concatenate([col, pc])
    rp2 = rowp * 2
    rows2 = jnp.concatenate([rp2, rp2 + 1])
    cols2 = colp
    xp = jnp.pad(x, ((0, NPAD - N), (0, 0)))
    Wt = jnp.transpose(W, (0, 2, 1))
    scale = 1.0 / jnp.sqrt(1.0 + EPS)
    gs = gamma * scale
    b8 = jnp.broadcast_to(b[:, None, :], (L, 8, D))
    gs8 = jnp.broadcast_to(gs[:, None, :], (L, 8, D))
    beta8 = jnp.broadcast_to(beta[:, None, :], (L, 8, D))
    wof = jnp.zeros((D, 128), jnp.float32).at[:, :OUT].set(W_out.T)
    bof = jnp.broadcast_to(jnp.pad(b_out, (0, 128 - OUT))[None, :], (8, 128))

    ones128 = jnp.ones((CH, HW), jnp.float32)
    zeroshw = jnp.zeros((ZR, HW), jnp.float32)

    deg2 = _sc_degree(colp, ones128, zeroshw)

    par_spec = _full_spec((8, D))

    def seg(z):
        return _sc_segsum(z.reshape(2 * NPAD, HW), rows2, cols2, zeroshw)

    z = _pcall(_tc_first_body,
               [_row_spec(), _full_spec((D, D)), _accl_spec, _accr_spec],
               (D,))(xp, Wt[0], deg2, deg2)
    h = xp
    for i in range(1, L):
        accs = seg(z)
        h, z = _pcall(_tc_mid_body,
                      [_accl_spec, _accr_spec, _row_spec(), _row_spec(),
                       _full_spec((D, D)), _accl_spec, _accr_spec,
                       par_spec, par_spec, par_spec], (D, D))(
            accs, accs, z, h, Wt[i], deg2, deg2,
            b8[i - 1], gs8[i - 1], beta8[i - 1])
    accs = seg(z)
    z = _pcall(_tc_last_body,
               [_accl_spec, _accr_spec, _row_spec(), _row_spec(),
                _accl_spec, _accr_spec,
                par_spec, par_spec, par_spec], (D,))(
        accs, accs, z, h, deg2, deg2, b8[L - 1], gs8[L - 1], beta8[L - 1])
    accs = seg(z)
    dxp = _pcall(_tc_out_body,
                 [_accl_spec, _accr_spec, _row_spec(), _accl_spec, _accr_spec,
                  _full_spec((D, 128)), _full_spec((8, 128))], (128,))(
        accs, accs, z, deg2, deg2, wof, bof)
    return dxp[:N, :OUT]
